# baseline (speedup 1.0000x reference)
import functools

import jax
import jax.numpy as jnp
from jax import lax
from jax.experimental import pallas as pl
from jax.experimental.pallas import tpu as pltpu
from jax.experimental.pallas import tpu_sc as plsc

VOCAB = 100000
EMBED_DIM = 16
BATCH = 16384

_INFO = plsc.get_sparse_core_info()
_NC = _INFO.num_cores
_NS = _INFO.num_subcores
_L = _INFO.num_lanes
_B_HALF = BATCH // _NC

_MESH = plsc.VectorSubcoreMesh(core_axis_name="c", subcore_axis_name="s")


@functools.partial(
    pl.kernel,
    mesh=_MESH,
    out_type=jax.ShapeDtypeStruct((EMBED_DIM, BATCH), jnp.float32),
    scratch_types=[
        pltpu.VMEM((VOCAB + 1,), jnp.float32),
        pltpu.VMEM((_B_HALF,), jnp.int32),
        pltpu.VMEM((_B_HALF,), jnp.float32),
        pltpu.SemaphoreType.DMA,
        pltpu.SemaphoreType.DMA,
    ],
    compiler_params=pltpu.CompilerParams(
        use_tc_tiling_on_sc=True, needs_layout_passes=False
    ),
)
def _embed_gather(idx_hbm, tableT_hbm, outT_hbm, slab_v, idx_v, out_v, sem_a, sem_b):
    c = lax.axis_index("c")
    s = lax.axis_index("s")
    cp_slab = pltpu.async_copy(tableT_hbm.at[s], slab_v, sem_a)
    cp_idx = pltpu.async_copy(idx_hbm.at[pl.ds(c * _B_HALF, _B_HALF)], idx_v, sem_b)
    cp_idx.wait()
    cp_slab.wait()

    half = _B_HALF // 2
    out_copies = []
    for h in range(2):

        @plsc.parallel_loop(h * (half // _L), (h + 1) * (half // _L), unroll=16)
        def _(i):
            sl = pl.ds(i * _L, _L)
            iv = idx_v[sl] + 1
            out_v[sl] = plsc.load_gather(slab_v, [iv])

        out_copies.append(
            pltpu.async_copy(
                out_v.at[pl.ds(h * half, half)],
                outT_hbm.at[s, pl.ds(c * _B_HALF + h * half, half)],
                sem_b,
            )
        )
    for cp in out_copies:
        cp.wait()


def kernel(source_id, table):
    outT = _embed_gather(source_id.astype(jnp.int32), table.T)
    return outT.T


# speedup vs baseline: 1.0026x; 1.0026x over previous
import functools

import jax
import jax.numpy as jnp
from jax import lax
from jax.experimental import pallas as pl
from jax.experimental.pallas import tpu as pltpu
from jax.experimental.pallas import tpu_sc as plsc

VOCAB = 100000
EMBED_DIM = 16
BATCH = 16384

_INFO = plsc.get_sparse_core_info()
_NC = _INFO.num_cores
_NS = _INFO.num_subcores
_L = _INFO.num_lanes
_B_HALF = BATCH // _NC

_MESH = plsc.VectorSubcoreMesh(core_axis_name="c", subcore_axis_name="s")


@functools.partial(
    pl.kernel,
    mesh=_MESH,
    out_type=jax.ShapeDtypeStruct((EMBED_DIM, BATCH), jnp.float32),
    scratch_types=[
        pltpu.VMEM((VOCAB + 1,), jnp.float32),
        pltpu.VMEM((_B_HALF,), jnp.int32),
        pltpu.VMEM((_B_HALF,), jnp.float32),
        pltpu.SemaphoreType.DMA,
        pltpu.SemaphoreType.DMA,
    ],
    compiler_params=pltpu.CompilerParams(
        use_tc_tiling_on_sc=True, needs_layout_passes=False
    ),
)
def _embed_gather(idx_hbm, tableT_hbm, outT_hbm, slab_v, idx_v, out_v, sem_a, sem_b):
    c = lax.axis_index("c")
    s = lax.axis_index("s")
    cp_slab = pltpu.async_copy(tableT_hbm.at[s], slab_v, sem_a)
    cp_idx = pltpu.async_copy(idx_hbm.at[pl.ds(c * _B_HALF, _B_HALF)], idx_v, sem_b)
    cp_idx.wait()
    cp_slab.wait()

    half = _B_HALF // 2
    out_copies = []
    for h in range(2):

        @plsc.parallel_loop(h * (half // _L), (h + 1) * (half // _L), unroll=8)
        def _(i):
            sl = pl.ds(i * _L, _L)
            iv = idx_v[sl] + 1
            out_v[sl] = plsc.load_gather(slab_v, [iv])

        out_copies.append(
            pltpu.async_copy(
                out_v.at[pl.ds(h * half, half)],
                outT_hbm.at[s, pl.ds(c * _B_HALF + h * half, half)],
                sem_b,
            )
        )
    for cp in out_copies:
        cp.wait()


def kernel(source_id, table):
    outT = _embed_gather(source_id.astype(jnp.int32), table.T)
    return outT.T


# speedup vs baseline: 1.0585x; 1.0558x over previous
import functools

import jax
import jax.numpy as jnp
from jax import lax
from jax.experimental import pallas as pl
from jax.experimental.pallas import tpu as pltpu
from jax.experimental.pallas import tpu_sc as plsc

VOCAB = 100000
EMBED_DIM = 16
BATCH = 16384

_INFO = plsc.get_sparse_core_info()
_NC = _INFO.num_cores
_NS = _INFO.num_subcores
_L = _INFO.num_lanes
_B_HALF = BATCH // _NC

_MESH = plsc.VectorSubcoreMesh(core_axis_name="c", subcore_axis_name="s")


@functools.partial(
    pl.kernel,
    mesh=_MESH,
    out_type=jax.ShapeDtypeStruct((EMBED_DIM, BATCH), jnp.float32),
    scratch_types=[
        pltpu.VMEM((VOCAB + 1,), jnp.float32),
        pltpu.VMEM((_B_HALF,), jnp.int32),
        pltpu.VMEM((_B_HALF,), jnp.float32),
        pltpu.VMEM_SHARED((_B_HALF,), jnp.int32),
        pltpu.SemaphoreType.DMA,
        pltpu.SemaphoreType.DMA,
    ],
    compiler_params=pltpu.CompilerParams(
        use_tc_tiling_on_sc=True, needs_layout_passes=False
    ),
)
def _embed_gather(
    idx_hbm, tableT_hbm, outT_hbm, slab_v, idx_v, out_v, idx_sh, sem_a, sem_b
):
    c = lax.axis_index("c")
    s = lax.axis_index("s")
    cp_slab = pltpu.async_copy(tableT_hbm.at[s], slab_v, sem_a)

    @pl.when(s == 0)
    def _():
        pltpu.sync_copy(idx_hbm.at[pl.ds(c * _B_HALF, _B_HALF)], idx_sh)

    plsc.subcore_barrier()
    pltpu.sync_copy(idx_sh, idx_v)
    cp_slab.wait()

    half = _B_HALF // 2
    out_copies = []
    for h in range(2):

        @plsc.parallel_loop(h * (half // _L), (h + 1) * (half // _L), unroll=8)
        def _(i):
            sl = pl.ds(i * _L, _L)
            iv = idx_v[sl] + 1
            out_v[sl] = plsc.load_gather(slab_v, [iv])

        out_copies.append(
            pltpu.async_copy(
                out_v.at[pl.ds(h * half, half)],
                outT_hbm.at[s, pl.ds(c * _B_HALF + h * half, half)],
                sem_b,
            )
        )
    for cp in out_copies:
        cp.wait()


def kernel(source_id, table):
    outT = _embed_gather(source_id.astype(jnp.int32), table.T)
    return outT.T
